# baseline (speedup 1.0000x reference)
import functools

import jax
import jax.numpy as jnp
from jax import lax
from jax.experimental import pallas as pl
from jax.experimental.pallas import tpu as pltpu
from jax.experimental.pallas import tpu_sc as plsc

EMBED = 64
SCALE = 8.0
NW = 32
CHUNK = 128
NB = 4
LANES = 16


@functools.lru_cache(maxsize=None)
def _build(n_tokens: int):
    b_per_w = n_tokens // NW
    nchunks = b_per_w // CHUNK
    assert nchunks % NB == 0

    mesh = plsc.VectorSubcoreMesh(core_axis_name="c", subcore_axis_name="s")

    @functools.partial(
        pl.kernel,
        mesh=mesh,
        compiler_params=pltpu.CompilerParams(use_tc_tiling_on_sc=False),
        out_type=jax.ShapeDtypeStruct((NW, nchunks, CHUNK, EMBED), jnp.float32),
        scratch_types=(
            [pltpu.VMEM((nchunks, CHUNK), jnp.int32)]
            + [pltpu.VMEM((CHUNK, EMBED), jnp.float32) for _ in range(2 * NB)]
            + [pltpu.SemaphoreType.DMA for _ in range(2 * NB)]
        ),
    )
    def emb_kernel(table_hbm, idx_hbm, out_hbm, idx_v, *rest):
        rows_in = rest[0:NB]
        rows_out = rest[NB:2 * NB]
        gsem = rest[2 * NB:3 * NB]
        osem = rest[3 * NB:4 * NB]

        wid = lax.axis_index("s") * 2 + lax.axis_index("c")

        pltpu.sync_copy(idx_hbm.at[wid], idx_v)

        for b in range(NB):
            pltpu.async_copy(table_hbm.at[idx_v.at[b]], rows_in[b], gsem[b])

        def outer(g, carry):
            for b in range(NB):
                j = g * NB + b
                pltpu.make_async_copy(
                    table_hbm.at[idx_v.at[j]], rows_in[b], gsem[b]).wait()

                @pl.when(j >= NB)
                def _():
                    pltpu.make_async_copy(
                        rows_out[b], out_hbm.at[wid, j - NB], osem[b]).wait()

                def srow(r, c2):
                    for c in range(EMBED // LANES):
                        rows_out[b][r, pl.ds(c * LANES, LANES)] = (
                            rows_in[b][r, pl.ds(c * LANES, LANES)] * SCALE)
                    return c2
                lax.fori_loop(0, CHUNK, srow, 0, unroll=2)

                @pl.when(j + NB < nchunks)
                def _():
                    pltpu.async_copy(
                        table_hbm.at[idx_v.at[j + NB]], rows_in[b], gsem[b])

                pltpu.async_copy(rows_out[b], out_hbm.at[wid, j], osem[b])
            return carry

        lax.fori_loop(0, nchunks // NB, outer, 0)

        for b in range(NB):
            pltpu.make_async_copy(
                rows_out[b], out_hbm.at[wid, nchunks - NB + b], osem[b]).wait()

    return emb_kernel


@jax.jit
def kernel(x, table):
    rows, cols = x.shape
    n = rows * cols
    xf = x.reshape(n).astype(jnp.int32)
    pad = (-n) % (NW * CHUNK)
    if pad:
        xf = jnp.concatenate([xf, jnp.zeros((pad,), jnp.int32)])
    nt = n + pad
    xf = xf.reshape(NW, nt // (NW * CHUNK), CHUNK)
    out = _build(nt)(table, xf)
    out = out.reshape(nt, EMBED)
    if pad:
        out = out[:n]
    return out.reshape(rows, cols, EMBED)


# speedup vs baseline: 1.2670x; 1.2670x over previous
import functools

import jax
import jax.numpy as jnp
from jax import lax
from jax.experimental import pallas as pl
from jax.experimental.pallas import tpu as pltpu
from jax.experimental.pallas import tpu_sc as plsc

EMBED = 64
SCALE = 8.0
NW = 32
NB = 2
LANES = 16


@functools.lru_cache(maxsize=None)
def _build(n_rows: int, n_cols: int):
    rows_per_w = n_rows // NW
    assert rows_per_w % NB == 0

    mesh = plsc.VectorSubcoreMesh(core_axis_name="c", subcore_axis_name="s")

    @functools.partial(
        pl.kernel,
        mesh=mesh,
        compiler_params=pltpu.CompilerParams(use_tc_tiling_on_sc=False),
        out_type=jax.ShapeDtypeStruct((n_rows, n_cols, EMBED), jnp.float32),
        scratch_types=(
            [pltpu.VMEM((rows_per_w, n_cols), jnp.int32)]
            + [pltpu.VMEM((n_cols, EMBED), jnp.float32) for _ in range(2 * NB)]
            + [pltpu.SemaphoreType.DMA for _ in range(2 * NB)]
        ),
    )
    def emb_kernel(table_hbm, idx_hbm, out_hbm, idx_v, *rest):
        rows_in = rest[0:NB]
        rows_out = rest[NB:2 * NB]
        gsem = rest[2 * NB:3 * NB]
        osem = rest[3 * NB:4 * NB]

        wid = lax.axis_index("s") * 2 + lax.axis_index("c")
        row0 = wid * rows_per_w

        pltpu.sync_copy(idx_hbm.at[pl.ds(row0, rows_per_w)], idx_v)

        for b in range(NB):
            pltpu.async_copy(table_hbm.at[idx_v.at[b]], rows_in[b], gsem[b])

        def outer(g, carry):
            for b in range(NB):
                j = g * NB + b
                pltpu.make_async_copy(
                    table_hbm.at[idx_v.at[j]], rows_in[b], gsem[b]).wait()

                @pl.when(j >= NB)
                def _():
                    pltpu.make_async_copy(
                        rows_out[b], out_hbm.at[row0 + j - NB], osem[b]).wait()

                @plsc.parallel_loop(0, n_cols, unroll=4)
                def srow(r):
                    for c in range(EMBED // LANES):
                        rows_out[b][r, pl.ds(c * LANES, LANES)] = (
                            rows_in[b][r, pl.ds(c * LANES, LANES)] * SCALE)

                @pl.when(j + NB < rows_per_w)
                def _():
                    pltpu.async_copy(
                        table_hbm.at[idx_v.at[j + NB]], rows_in[b], gsem[b])

                pltpu.async_copy(rows_out[b], out_hbm.at[row0 + j], osem[b])
            return carry

        lax.fori_loop(0, rows_per_w // NB, outer, 0)

        for b in range(NB):
            pltpu.make_async_copy(
                rows_out[b],
                out_hbm.at[row0 + rows_per_w - NB + b], osem[b]).wait()

    return emb_kernel


@jax.jit
def kernel(x, table):
    n_rows, n_cols = x.shape
    xi = x.astype(jnp.int32)
    pad = (-n_rows) % (NW * NB)
    if pad:
        xi = jnp.concatenate([xi, jnp.zeros((pad, n_cols), jnp.int32)])
    out = _build(n_rows + pad, n_cols)(table, xi)
    if pad:
        out = out[:n_rows]
    return out
